# baseline (device time: 60556 ns/iter reference)
import jax
import jax.numpy as jnp
from jax import lax
from jax.experimental import pallas as pl
from jax.experimental.pallas import tpu as pltpu

N_DEV = 16
HOPS = 7
SUB = 2


def _gelu_f32(y):
    c = 0.7978845608028654
    return 0.5 * y * (1.0 + jnp.tanh(c * (y + 0.044715 * y * y * y)))


def kernel(x, w_mat):
    m_per, k = x.shape
    _, n_per = w_mat.shape
    msub = m_per // SUB

    def body(x_ref, w_ref, out_ref, comm_ref, w_bf, send_r, recv_r,
             send_l, recv_l, send_a, recv_a):
        my = lax.axis_index("i")
        left = (my - 1) % N_DEV
        right = (my + 1) % N_DEV
        anti = (my + 8) % N_DEV

        barrier_sem = pltpu.get_barrier_semaphore()
        for nbr in (left, right, anti):
            pl.semaphore_signal(
                barrier_sem, inc=1,
                device_id=(nbr,), device_id_type=pl.DeviceIdType.MESH,
            )
        pl.semaphore_wait(barrier_sem, 3)

        comm_ref[0, :, :] = x_ref[:, :].astype(jnp.bfloat16)

        sends = []

        def _send(src_slot, dst_slot, j, ssem, rsem, dst_dev):
            rows = pl.ds(j * msub, msub)
            rdma = pltpu.make_async_remote_copy(
                src_ref=comm_ref.at[src_slot, rows],
                dst_ref=comm_ref.at[dst_slot, rows],
                send_sem=ssem, recv_sem=rsem,
                device_id=(dst_dev,),
                device_id_type=pl.DeviceIdType.MESH,
            )
            rdma.start()
            sends.append(rdma)

        def _wait_recv(dst_slot, j, ssem, rsem):
            rows = pl.ds(j * msub, msub)
            rdma = pltpu.make_async_remote_copy(
                src_ref=comm_ref.at[dst_slot, rows],
                dst_ref=comm_ref.at[dst_slot, rows],
                send_sem=ssem, recv_sem=rsem,
                device_id=(left,), device_id_type=pl.DeviceIdType.MESH,
            )
            rdma.wait_recv()

        for j in range(SUB):
            _send(0, 1, j, send_r.at[0, j], recv_r.at[0, j], right)
            _send(0, 9, j, send_l.at[0, j], recv_l.at[0, j], left)

        anti_rdma = pltpu.make_async_remote_copy(
            src_ref=comm_ref.at[0], dst_ref=comm_ref.at[8],
            send_sem=send_a.at[0], recv_sem=recv_a.at[0],
            device_id=(anti,), device_id_type=pl.DeviceIdType.MESH,
        )
        sends.append(anti_rdma)
        q = my % 4
        idle_col = jnp.logical_or(q == 1, q == 2)

        @pl.when(idle_col)
        def _():
            anti_rdma.start()

        w_bf[:, :] = w_ref[:, :].astype(jnp.bfloat16)
        y0 = jnp.dot(comm_ref[0, :, :], w_bf[:, :],
                     preferred_element_type=jnp.float32)
        out_ref[pl.ds(my * m_per, m_per), :] = _gelu_f32(y0)

        for s in range(HOPS):
            for j in range(SUB):
                _wait_recv(s + 1, j, send_r.at[s, j], recv_r.at[s, j])
                if s + 1 < HOPS:
                    _send(s + 1, s + 2, j,
                          send_r.at[s + 1, j], recv_r.at[s + 1, j], right)
                _wait_recv(9 + s, j, send_l.at[s, j], recv_l.at[s, j])
                if s + 1 < HOPS:
                    _send(9 + s, 9 + s + 1, j,
                          send_l.at[s + 1, j], recv_l.at[s + 1, j], left)

            if s == 4:
                @pl.when(jnp.logical_not(idle_col))
                def _():
                    anti_rdma.start()

            origin_r = (my - s - 1) % N_DEV
            yr = jnp.dot(comm_ref[s + 1, :, :], w_bf[:, :],
                         preferred_element_type=jnp.float32)
            out_ref[pl.ds(origin_r * m_per, m_per), :] = _gelu_f32(yr)
            origin_l = (my + s + 1) % N_DEV
            yl = jnp.dot(comm_ref[9 + s, :, :], w_bf[:, :],
                         preferred_element_type=jnp.float32)
            out_ref[pl.ds(origin_l * m_per, m_per), :] = _gelu_f32(yl)

        anti_rdma.wait_recv()
        ya = jnp.dot(comm_ref[8, :, :], w_bf[:, :],
                     preferred_element_type=jnp.float32)
        out_ref[pl.ds(anti * m_per, m_per), :] = _gelu_f32(ya)

        for rdma in sends:
            rdma.wait_send()

    return pl.pallas_call(
        body,
        out_shape=jax.ShapeDtypeStruct((N_DEV * m_per, n_per), jnp.float32),
        in_specs=[
            pl.BlockSpec(memory_space=pltpu.VMEM),
            pl.BlockSpec(memory_space=pltpu.VMEM),
        ],
        out_specs=pl.BlockSpec(memory_space=pltpu.VMEM),
        scratch_shapes=[
            pltpu.VMEM((N_DEV, m_per, k), jnp.bfloat16),
            pltpu.VMEM((k, n_per), jnp.bfloat16),
            pltpu.SemaphoreType.DMA((HOPS, SUB)),
            pltpu.SemaphoreType.DMA((HOPS, SUB)),
            pltpu.SemaphoreType.DMA((HOPS, SUB)),
            pltpu.SemaphoreType.DMA((HOPS, SUB)),
            pltpu.SemaphoreType.DMA((1,)),
            pltpu.SemaphoreType.DMA((1,)),
        ],
        compiler_params=pltpu.CompilerParams(collective_id=0),
    )(x, w_mat)


# device time: 59600 ns/iter; 1.0160x vs baseline; 1.0160x over previous
import jax
import jax.numpy as jnp
from jax import lax
from jax.experimental import pallas as pl
from jax.experimental.pallas import tpu as pltpu

N_DEV = 16
HOPS = 7
SUB = 2


def _gelu_f32(y):
    c = 0.7978845608028654
    return 0.5 * y * (1.0 + jnp.tanh(c * (y + 0.044715 * y * y * y)))


def kernel(x, w_mat):
    m_per, k = x.shape
    _, n_per = w_mat.shape
    msub = m_per // SUB

    def body(x_ref, w_ref, out_ref, comm_ref, w_bf, send_r, recv_r,
             send_l, recv_l, send_a, recv_a):
        my = lax.axis_index("i")
        left = (my - 1) % N_DEV
        right = (my + 1) % N_DEV
        anti = (my + 8) % N_DEV

        barrier_sem = pltpu.get_barrier_semaphore()
        for nbr in (left, right, anti):
            pl.semaphore_signal(
                barrier_sem, inc=1,
                device_id=(nbr,), device_id_type=pl.DeviceIdType.MESH,
            )
        pl.semaphore_wait(barrier_sem, 3)

        comm_ref[0, :, :] = x_ref[:, :].astype(jnp.bfloat16)

        sends = []

        def _send(src_slot, dst_slot, j, ssem, rsem, dst_dev):
            rows = pl.ds(j * msub, msub)
            rdma = pltpu.make_async_remote_copy(
                src_ref=comm_ref.at[src_slot, rows],
                dst_ref=comm_ref.at[dst_slot, rows],
                send_sem=ssem, recv_sem=rsem,
                device_id=(dst_dev,),
                device_id_type=pl.DeviceIdType.MESH,
            )
            rdma.start()
            sends.append(rdma)

        def _wait_recv(dst_slot, j, ssem, rsem):
            rows = pl.ds(j * msub, msub)
            rdma = pltpu.make_async_remote_copy(
                src_ref=comm_ref.at[dst_slot, rows],
                dst_ref=comm_ref.at[dst_slot, rows],
                send_sem=ssem, recv_sem=rsem,
                device_id=(left,), device_id_type=pl.DeviceIdType.MESH,
            )
            rdma.wait_recv()

        for j in range(SUB):
            _send(0, 1, j, send_r.at[0, j], recv_r.at[0, j], right)
            _send(0, 9, j, send_l.at[0, j], recv_l.at[0, j], left)

        anti_rdma = pltpu.make_async_remote_copy(
            src_ref=comm_ref.at[0], dst_ref=comm_ref.at[8],
            send_sem=send_a.at[0], recv_sem=recv_a.at[0],
            device_id=(anti,), device_id_type=pl.DeviceIdType.MESH,
        )
        sends.append(anti_rdma)
        anti_rdma.start()

        w_bf[:, :] = w_ref[:, :].astype(jnp.bfloat16)
        y0 = jnp.dot(comm_ref[0, :, :], w_bf[:, :],
                     preferred_element_type=jnp.float32)
        out_ref[pl.ds(my * m_per, m_per), :] = _gelu_f32(y0)

        for s in range(HOPS):
            for j in range(SUB):
                _wait_recv(s + 1, j, send_r.at[s, j], recv_r.at[s, j])
                if s + 1 < HOPS:
                    _send(s + 1, s + 2, j,
                          send_r.at[s + 1, j], recv_r.at[s + 1, j], right)
                _wait_recv(9 + s, j, send_l.at[s, j], recv_l.at[s, j])
                if s + 1 < HOPS:
                    _send(9 + s, 9 + s + 1, j,
                          send_l.at[s + 1, j], recv_l.at[s + 1, j], left)

            origin_r = (my - s - 1) % N_DEV
            yr = jnp.dot(comm_ref[s + 1, :, :], w_bf[:, :],
                         preferred_element_type=jnp.float32)
            out_ref[pl.ds(origin_r * m_per, m_per), :] = _gelu_f32(yr)
            origin_l = (my + s + 1) % N_DEV
            yl = jnp.dot(comm_ref[9 + s, :, :], w_bf[:, :],
                         preferred_element_type=jnp.float32)
            out_ref[pl.ds(origin_l * m_per, m_per), :] = _gelu_f32(yl)

        anti_rdma.wait_recv()
        ya = jnp.dot(comm_ref[8, :, :], w_bf[:, :],
                     preferred_element_type=jnp.float32)
        out_ref[pl.ds(anti * m_per, m_per), :] = _gelu_f32(ya)

        for rdma in sends:
            rdma.wait_send()

    return pl.pallas_call(
        body,
        out_shape=jax.ShapeDtypeStruct((N_DEV * m_per, n_per), jnp.float32),
        in_specs=[
            pl.BlockSpec(memory_space=pltpu.VMEM),
            pl.BlockSpec(memory_space=pltpu.VMEM),
        ],
        out_specs=pl.BlockSpec(memory_space=pltpu.VMEM),
        scratch_shapes=[
            pltpu.VMEM((N_DEV, m_per, k), jnp.bfloat16),
            pltpu.VMEM((k, n_per), jnp.bfloat16),
            pltpu.SemaphoreType.DMA((HOPS, SUB)),
            pltpu.SemaphoreType.DMA((HOPS, SUB)),
            pltpu.SemaphoreType.DMA((HOPS, SUB)),
            pltpu.SemaphoreType.DMA((HOPS, SUB)),
            pltpu.SemaphoreType.DMA((1,)),
            pltpu.SemaphoreType.DMA((1,)),
        ],
        compiler_params=pltpu.CompilerParams(collective_id=0),
    )(x, w_mat)


# device time: 55450 ns/iter; 1.0921x vs baseline; 1.0748x over previous
import jax
import jax.numpy as jnp
from jax import lax
from jax.experimental import pallas as pl
from jax.experimental.pallas import tpu as pltpu

N_DEV = 16
HOPS = 8
SUB = 2


def _gelu_f32(y):
    c = 0.7978845608028654
    return 0.5 * y * (1.0 + jnp.tanh(c * (y + 0.044715 * y * y * y)))


def kernel(x, w_mat):
    m_per, k = x.shape
    _, n_per = w_mat.shape
    msub = m_per // SUB

    def _r_active(s, j):
        return s < HOPS - 1 or j == 0

    def _l_active(s, j):
        return s < HOPS - 1 or j == 1

    def _r_dst(s):
        return s + 1

    def _l_dst(s):
        return 8 if s == HOPS - 1 else 9 + s

    def _r_src(s):
        return 0 if s == 0 else s

    def _l_src(s):
        return 0 if s == 0 else 8 + s

    def body(x_ref, w_ref, out_ref, comm_ref, w_bf, send_r, recv_r,
             send_l, recv_l):
        my = lax.axis_index("i")
        left = (my - 1) % N_DEV
        right = (my + 1) % N_DEV

        barrier_sem = pltpu.get_barrier_semaphore()
        for nbr in (left, right):
            pl.semaphore_signal(
                barrier_sem, inc=1,
                device_id=(nbr,), device_id_type=pl.DeviceIdType.MESH,
            )
        pl.semaphore_wait(barrier_sem, 2)

        comm_ref[0, :, :] = x_ref[:, :].astype(jnp.bfloat16)

        sends = []

        def _send(src_slot, dst_slot, j, ssem, rsem, dst_dev):
            rows = pl.ds(j * msub, msub)
            rdma = pltpu.make_async_remote_copy(
                src_ref=comm_ref.at[src_slot, rows],
                dst_ref=comm_ref.at[dst_slot, rows],
                send_sem=ssem, recv_sem=rsem,
                device_id=(dst_dev,),
                device_id_type=pl.DeviceIdType.MESH,
            )
            rdma.start()
            sends.append(rdma)

        def _wait_recv(dst_slot, j, ssem, rsem):
            rows = pl.ds(j * msub, msub)
            rdma = pltpu.make_async_remote_copy(
                src_ref=comm_ref.at[dst_slot, rows],
                dst_ref=comm_ref.at[dst_slot, rows],
                send_sem=ssem, recv_sem=rsem,
                device_id=(left,), device_id_type=pl.DeviceIdType.MESH,
            )
            rdma.wait_recv()

        for j in range(SUB):
            _send(0, _r_dst(0), j, send_r.at[0, j], recv_r.at[0, j], right)
            _send(0, _l_dst(0), j, send_l.at[0, j], recv_l.at[0, j], left)

        w_bf[:, :] = w_ref[:, :].astype(jnp.bfloat16)
        y0 = jnp.dot(comm_ref[0, :, :], w_bf[:, :],
                     preferred_element_type=jnp.float32)
        out_ref[pl.ds(my * m_per, m_per), :] = _gelu_f32(y0)

        for s in range(HOPS):
            for j in range(SUB):
                if _r_active(s, j):
                    _wait_recv(_r_dst(s), j, send_r.at[s, j], recv_r.at[s, j])
                    if s + 1 < HOPS and _r_active(s + 1, j):
                        _send(_r_src(s + 1), _r_dst(s + 1), j,
                              send_r.at[s + 1, j], recv_r.at[s + 1, j], right)
                if _l_active(s, j):
                    _wait_recv(_l_dst(s), j, send_l.at[s, j], recv_l.at[s, j])
                    if s + 1 < HOPS and _l_active(s + 1, j):
                        _send(_l_src(s + 1), _l_dst(s + 1), j,
                              send_l.at[s + 1, j], recv_l.at[s + 1, j], left)

            if s < HOPS - 1:
                origin_r = (my - s - 1) % N_DEV
                yr = jnp.dot(comm_ref[s + 1, :, :], w_bf[:, :],
                             preferred_element_type=jnp.float32)
                out_ref[pl.ds(origin_r * m_per, m_per), :] = _gelu_f32(yr)
                origin_l = (my + s + 1) % N_DEV
                yl = jnp.dot(comm_ref[9 + s, :, :], w_bf[:, :],
                             preferred_element_type=jnp.float32)
                out_ref[pl.ds(origin_l * m_per, m_per), :] = _gelu_f32(yl)
            else:
                origin_a = (my + HOPS) % N_DEV
                ya = jnp.dot(comm_ref[8, :, :], w_bf[:, :],
                             preferred_element_type=jnp.float32)
                out_ref[pl.ds(origin_a * m_per, m_per), :] = _gelu_f32(ya)

        for rdma in sends:
            rdma.wait_send()

    return pl.pallas_call(
        body,
        out_shape=jax.ShapeDtypeStruct((N_DEV * m_per, n_per), jnp.float32),
        in_specs=[
            pl.BlockSpec(memory_space=pltpu.VMEM),
            pl.BlockSpec(memory_space=pltpu.VMEM),
        ],
        out_specs=pl.BlockSpec(memory_space=pltpu.VMEM),
        scratch_shapes=[
            pltpu.VMEM((N_DEV, m_per, k), jnp.bfloat16),
            pltpu.VMEM((k, n_per), jnp.bfloat16),
            pltpu.SemaphoreType.DMA((HOPS, SUB)),
            pltpu.SemaphoreType.DMA((HOPS, SUB)),
            pltpu.SemaphoreType.DMA((HOPS, SUB)),
            pltpu.SemaphoreType.DMA((HOPS, SUB)),
        ],
        compiler_params=pltpu.CompilerParams(collective_id=0),
    )(x, w_mat)


# device time: 54835 ns/iter; 1.1043x vs baseline; 1.0112x over previous
import jax
import jax.numpy as jnp
from jax import lax
from jax.experimental import pallas as pl
from jax.experimental.pallas import tpu as pltpu

N_DEV = 16
HOPS = 8
SUB = 4


def _gelu_f32(y):
    c = 0.7978845608028654
    return 0.5 * y * (1.0 + jnp.tanh(c * (y + 0.044715 * y * y * y)))


def kernel(x, w_mat):
    m_per, k = x.shape
    _, n_per = w_mat.shape
    msub = m_per // SUB

    def _r_active(s, j):
        return s < HOPS - 1 or j < SUB // 2

    def _l_active(s, j):
        return s < HOPS - 1 or j >= SUB // 2

    def _r_dst(s):
        return s + 1

    def _l_dst(s):
        return 8 if s == HOPS - 1 else 9 + s

    def _r_src(s):
        return 0 if s == 0 else s

    def _l_src(s):
        return 0 if s == 0 else 8 + s

    def body(x_ref, w_ref, out_ref, comm_ref, w_bf, send_r, recv_r,
             send_l, recv_l):
        my = lax.axis_index("i")
        left = (my - 1) % N_DEV
        right = (my + 1) % N_DEV

        barrier_sem = pltpu.get_barrier_semaphore()
        for nbr in (left, right):
            pl.semaphore_signal(
                barrier_sem, inc=1,
                device_id=(nbr,), device_id_type=pl.DeviceIdType.MESH,
            )
        pl.semaphore_wait(barrier_sem, 2)

        comm_ref[0, :, :] = x_ref[:, :].astype(jnp.bfloat16)

        sends = []

        def _send(src_slot, dst_slot, j, ssem, rsem, dst_dev):
            rows = pl.ds(j * msub, msub)
            rdma = pltpu.make_async_remote_copy(
                src_ref=comm_ref.at[src_slot, rows],
                dst_ref=comm_ref.at[dst_slot, rows],
                send_sem=ssem, recv_sem=rsem,
                device_id=(dst_dev,),
                device_id_type=pl.DeviceIdType.MESH,
            )
            rdma.start()
            sends.append(rdma)

        def _wait_recv(dst_slot, j, ssem, rsem):
            rows = pl.ds(j * msub, msub)
            rdma = pltpu.make_async_remote_copy(
                src_ref=comm_ref.at[dst_slot, rows],
                dst_ref=comm_ref.at[dst_slot, rows],
                send_sem=ssem, recv_sem=rsem,
                device_id=(left,), device_id_type=pl.DeviceIdType.MESH,
            )
            rdma.wait_recv()

        for j in range(SUB):
            _send(0, _r_dst(0), j, send_r.at[0, j], recv_r.at[0, j], right)
            _send(0, _l_dst(0), j, send_l.at[0, j], recv_l.at[0, j], left)

        w_bf[:, :] = w_ref[:, :].astype(jnp.bfloat16)
        y0 = jnp.dot(comm_ref[0, :, :], w_bf[:, :],
                     preferred_element_type=jnp.float32)
        out_ref[pl.ds(my * m_per, m_per), :] = _gelu_f32(y0)

        for s in range(HOPS):
            for j in range(SUB):
                if _r_active(s, j):
                    _wait_recv(_r_dst(s), j, send_r.at[s, j], recv_r.at[s, j])
                    if s + 1 < HOPS and _r_active(s + 1, j):
                        _send(_r_src(s + 1), _r_dst(s + 1), j,
                              send_r.at[s + 1, j], recv_r.at[s + 1, j], right)
                if _l_active(s, j):
                    _wait_recv(_l_dst(s), j, send_l.at[s, j], recv_l.at[s, j])
                    if s + 1 < HOPS and _l_active(s + 1, j):
                        _send(_l_src(s + 1), _l_dst(s + 1), j,
                              send_l.at[s + 1, j], recv_l.at[s + 1, j], left)

            if s < HOPS - 1:
                origin_r = (my - s - 1) % N_DEV
                yr = jnp.dot(comm_ref[s + 1, :, :], w_bf[:, :],
                             preferred_element_type=jnp.float32)
                out_ref[pl.ds(origin_r * m_per, m_per), :] = _gelu_f32(yr)
                origin_l = (my + s + 1) % N_DEV
                yl = jnp.dot(comm_ref[9 + s, :, :], w_bf[:, :],
                             preferred_element_type=jnp.float32)
                out_ref[pl.ds(origin_l * m_per, m_per), :] = _gelu_f32(yl)
            else:
                origin_a = (my + HOPS) % N_DEV
                ya = jnp.dot(comm_ref[8, :, :], w_bf[:, :],
                             preferred_element_type=jnp.float32)
                out_ref[pl.ds(origin_a * m_per, m_per), :] = _gelu_f32(ya)

        for rdma in sends:
            rdma.wait_send()

    return pl.pallas_call(
        body,
        out_shape=jax.ShapeDtypeStruct((N_DEV * m_per, n_per), jnp.float32),
        in_specs=[
            pl.BlockSpec(memory_space=pltpu.VMEM),
            pl.BlockSpec(memory_space=pltpu.VMEM),
        ],
        out_specs=pl.BlockSpec(memory_space=pltpu.VMEM),
        scratch_shapes=[
            pltpu.VMEM((N_DEV, m_per, k), jnp.bfloat16),
            pltpu.VMEM((k, n_per), jnp.bfloat16),
            pltpu.SemaphoreType.DMA((HOPS, SUB)),
            pltpu.SemaphoreType.DMA((HOPS, SUB)),
            pltpu.SemaphoreType.DMA((HOPS, SUB)),
            pltpu.SemaphoreType.DMA((HOPS, SUB)),
        ],
        compiler_params=pltpu.CompilerParams(collective_id=0),
    )(x, w_mat)
